# baseline (speedup 1.0000x reference)
import functools

import jax
import jax.numpy as jnp
from jax import lax
from jax.experimental import pallas as pl
from jax.experimental.pallas import tpu as pltpu
from jax.experimental.pallas import tpu_sc as plsc

_F = 100000
_NC, _NS, _L = 2, 16, 16
_NW = _NC * _NS
_CH = 3200
_NV = _CH // _L
_LAST = 96896
_UNROLL = 1


def _sc_body(int_ref, out_ref, xb, ov, sem):
    cid = lax.axis_index("c")
    sid = lax.axis_index("s")
    wid = sid * _NC + cid
    off = pl.multiple_of(jnp.minimum(wid * _CH, _LAST), 128)

    pltpu.async_copy(int_ref.at[:, pl.ds(off, _CH)], xb, sem).wait()

    def step(i, carry):
        for u in range(_UNROLL):
            sl = pl.ds((i * _UNROLL + u) * _L, _L)
            ov[0, sl] = xb[0, sl] + xb[1, sl] + xb[2, sl]
        return carry

    lax.fori_loop(0, _NV // _UNROLL, step, 0)
    pltpu.sync_copy(ov, out_ref.at[:, pl.ds(off, _CH)])


@functools.partial(
    pl.kernel,
    mesh=plsc.VectorSubcoreMesh(core_axis_name="c", subcore_axis_name="s"),
    out_type=jax.ShapeDtypeStruct((1, _F), jnp.float32),
    scratch_types=[
        pltpu.VMEM((3, _CH), jnp.float32),
        pltpu.VMEM((1, _CH), jnp.float32),
        pltpu.SemaphoreType.DMA,
    ],
)
def _sc_pool(int_ref, out_ref, xb, ov, sem):
    _sc_body(int_ref, out_ref, xb, ov, sem)


def kernel(intensities, avg_dist, W_gcn, b_gcn, W_attn, b_attn):
    return _sc_pool(intensities)
